# baseline (speedup 1.0000x reference)
import functools

import jax
import jax.numpy as jnp
import numpy as np
from jax import lax
from jax.experimental import pallas as pl
from jax.experimental.pallas import tpu as pltpu
from jax.experimental.pallas import tpu_sc as plsc

HIDDEN_SIZE = 2048
NUM_HEADS = 32
HEAD_DIM = HIDDEN_SIZE // NUM_HEADS
ROPE_THETA = 10000.0
MAX_POS = 4096
ATTENTION_SCALING = 1.0
BATCH = 2
SEQ = 4096

NUM_IDX = BATCH * SEQ
NC, NS = 2, 16
NW = NC * NS
IDX_PER_W = NUM_IDX // NW
CHUNK = 128
NCH = IDX_PER_W // CHUNK
FUSED = 2 * HEAD_DIM


def _build_fused_table():
    inv_freq = (1.0 / (ROPE_THETA ** (np.arange(0, HEAD_DIM, 2, dtype=np.float32) / HEAD_DIM))).astype(np.float32)
    t = np.arange(MAX_POS, dtype=np.float32)
    freqs = np.outer(t, inv_freq).astype(np.float32)
    emb = np.concatenate([freqs, freqs], axis=-1)
    cos = (np.cos(emb) * ATTENTION_SCALING).astype(np.float32)
    sin = (np.sin(emb) * ATTENTION_SCALING).astype(np.float32)
    return np.concatenate([cos, sin], axis=-1)


_FUSED_TABLE = _build_fused_table()


@functools.partial(
    pl.kernel,
    mesh=plsc.VectorSubcoreMesh(core_axis_name="c", subcore_axis_name="s"),
    out_type=jax.ShapeDtypeStruct((NUM_IDX, FUSED), jnp.float32),
    scratch_types=[
        pltpu.VMEM((NCH, CHUNK), jnp.int32),
        pltpu.VMEM((IDX_PER_W, FUSED), jnp.float32),
        pltpu.SemaphoreType.DMA,
        pltpu.SemaphoreType.DMA,
        pltpu.SemaphoreType.DMA,
    ],
    compiler_params=pltpu.CompilerParams(
        use_tc_tiling_on_sc=False,
        disable_bounds_checks=True,
        disable_semaphore_checks=True,
    ),
)
def _rope_gather(tab_hbm, idx_hbm, fused_out, idx_v, rows_v, sem_a, sem_b, sem_st):
    wid = lax.axis_index("s") * NC + lax.axis_index("c")
    base = wid * IDX_PER_W
    pltpu.sync_copy(idx_hbm.at[pl.ds(wid * NCH, NCH)], idx_v)
    gsems = (sem_a, sem_b)
    gathers = [
        pltpu.async_copy(tab_hbm.at[idx_v.at[j]],
                         rows_v.at[pl.ds(j * CHUNK, CHUNK)], gsems[j])
        for j in range(NCH)
    ]
    stores = []
    for j in range(NCH):
        gathers[j].wait()
        stores.append(pltpu.async_copy(
            rows_v.at[pl.ds(j * CHUNK, CHUNK)],
            fused_out.at[pl.ds(base + j * CHUNK, CHUNK)], sem_st))
    for st in stores:
        st.wait()


def _split_body(fused_ref, cos_ref, sin_ref):
    rows = fused_ref[...]
    cos_ref[...] = rows[None, :, :HEAD_DIM]
    sin_ref[...] = rows[None, :, HEAD_DIM:]


_ROWS_PER_TC_BLK = 1024
_TC_GRID = NUM_IDX // _ROWS_PER_TC_BLK
_BLKS_PER_B = SEQ // _ROWS_PER_TC_BLK


def _split_fused(fused):
    return pl.pallas_call(
        _split_body,
        grid=(_TC_GRID,),
        in_specs=[pl.BlockSpec((_ROWS_PER_TC_BLK, FUSED), lambda i: (i, 0))],
        out_specs=[
            pl.BlockSpec((1, _ROWS_PER_TC_BLK, HEAD_DIM),
                         lambda i: (i // _BLKS_PER_B, i % _BLKS_PER_B, 0)),
            pl.BlockSpec((1, _ROWS_PER_TC_BLK, HEAD_DIM),
                         lambda i: (i // _BLKS_PER_B, i % _BLKS_PER_B, 0)),
        ],
        out_shape=[
            jax.ShapeDtypeStruct((BATCH, SEQ, HEAD_DIM), jnp.float32),
            jax.ShapeDtypeStruct((BATCH, SEQ, HEAD_DIM), jnp.float32),
        ],
    )(fused)


def kernel(x, position_ids):
    tab = jnp.asarray(_FUSED_TABLE)
    idx = position_ids.reshape(NW * NCH, CHUNK)
    fused = _rope_gather(tab, idx)
    cos_o, sin_o = _split_fused(fused)
    return cos_o.astype(x.dtype), sin_o.astype(x.dtype)


# speedup vs baseline: 1.4587x; 1.4587x over previous
import functools

import jax
import jax.numpy as jnp
import numpy as np
from jax import lax
from jax.experimental import pallas as pl
from jax.experimental.pallas import tpu as pltpu
from jax.experimental.pallas import tpu_sc as plsc

HIDDEN_SIZE = 2048
NUM_HEADS = 32
HEAD_DIM = HIDDEN_SIZE // NUM_HEADS
ROPE_THETA = 10000.0
MAX_POS = 4096
ATTENTION_SCALING = 1.0
BATCH = 2
SEQ = 4096

NUM_IDX = BATCH * SEQ
NC, NS = 2, 16
NW = NC * NS
IDX_PER_W = NUM_IDX // NW
CHUNK = 128
NCH = IDX_PER_W // CHUNK


def _build_tables():
    inv_freq = (1.0 / (ROPE_THETA ** (np.arange(0, HEAD_DIM, 2, dtype=np.float32) / HEAD_DIM))).astype(np.float32)
    t = np.arange(MAX_POS, dtype=np.float32)
    freqs = np.outer(t, inv_freq).astype(np.float32)
    emb = np.concatenate([freqs, freqs], axis=-1)
    cos = (np.cos(emb) * ATTENTION_SCALING).astype(np.float32)
    sin = (np.sin(emb) * ATTENTION_SCALING).astype(np.float32)
    return cos, sin


_COS_TABLE, _SIN_TABLE = _build_tables()


@functools.partial(
    pl.kernel,
    mesh=plsc.VectorSubcoreMesh(core_axis_name="c", subcore_axis_name="s"),
    out_type=(
        jax.ShapeDtypeStruct((NUM_IDX, HEAD_DIM), jnp.float32),
        jax.ShapeDtypeStruct((NUM_IDX, HEAD_DIM), jnp.float32),
    ),
    scratch_types=[
        pltpu.VMEM((NCH, CHUNK), jnp.int32),
        pltpu.VMEM((IDX_PER_W, HEAD_DIM), jnp.float32),
        pltpu.VMEM((IDX_PER_W, HEAD_DIM), jnp.float32),
        pltpu.SemaphoreType.DMA,
        pltpu.SemaphoreType.DMA,
        pltpu.SemaphoreType.DMA,
    ],
    compiler_params=pltpu.CompilerParams(
        use_tc_tiling_on_sc=False,
        disable_bounds_checks=True,
        disable_semaphore_checks=True,
    ),
)
def _rope_gather(cos_hbm, sin_hbm, idx_hbm, cos_out, sin_out,
                 idx_v, cos_rows, sin_rows, sem_a, sem_b, sem_st):
    wid = lax.axis_index("s") * NC + lax.axis_index("c")
    base = wid * IDX_PER_W
    pltpu.sync_copy(idx_hbm.at[pl.ds(wid * NCH, NCH)], idx_v)
    gsems = (sem_a, sem_b)
    gathers = []
    for j in range(NCH):
        gathers.append((
            pltpu.async_copy(cos_hbm.at[idx_v.at[j]],
                             cos_rows.at[pl.ds(j * CHUNK, CHUNK)], gsems[j]),
            pltpu.async_copy(sin_hbm.at[idx_v.at[j]],
                             sin_rows.at[pl.ds(j * CHUNK, CHUNK)], gsems[j]),
        ))
    stores = []
    for j in range(NCH):
        gathers[j][0].wait()
        gathers[j][1].wait()
        stores.append(pltpu.async_copy(
            cos_rows.at[pl.ds(j * CHUNK, CHUNK)],
            cos_out.at[pl.ds(base + j * CHUNK, CHUNK)], sem_st))
        stores.append(pltpu.async_copy(
            sin_rows.at[pl.ds(j * CHUNK, CHUNK)],
            sin_out.at[pl.ds(base + j * CHUNK, CHUNK)], sem_st))
    for st in stores:
        st.wait()


def kernel(x, position_ids):
    cos_t = jnp.asarray(_COS_TABLE)
    sin_t = jnp.asarray(_SIN_TABLE)
    idx = position_ids.reshape(NW * NCH, CHUNK)
    cos_o, sin_o = _rope_gather(cos_t, sin_t, idx)
    cos_o = cos_o.reshape(BATCH, SEQ, HEAD_DIM).astype(x.dtype)
    sin_o = sin_o.reshape(BATCH, SEQ, HEAD_DIM).astype(x.dtype)
    return cos_o, sin_o
